# baseline (speedup 1.0000x reference)
import functools
import jax
import jax.numpy as jnp
from jax import lax
from jax.experimental import pallas as pl
from jax.experimental.pallas import tpu as pltpu
from jax.experimental.pallas import tpu_sc as plsc

B, H, W, C = 64, 32, 32, 768
N = B * H * W
NW = 32
R_SC = N
N_TC = N - R_SC
ROWS_PW = R_SC // NW
CH = 32
NCHUNK = ROWS_PW // CH
K = C // 16


def _sc_cam_body(x_hbm, w_hbm, p_hbm, wv, xb, ob, sem):
    wid = lax.axis_index("s") * 2 + lax.axis_index("c")
    base = N_TC + wid * ROWS_PW
    pltpu.sync_copy(w_hbm, wv)

    def chunk_src(g):
        return x_hbm.at[pl.ds((base + g * CH) * C, CH * C)]

    pltpu.make_async_copy(chunk_src(0), xb.at[0], sem.at[0]).start()

    def chunk_body(g, carry):
        slot = lax.rem(g, 2)
        nslot = lax.rem(g + 1, 2)
        pltpu.make_async_copy(chunk_src(g), xb.at[slot], sem.at[slot]).wait()

        @pl.when(g + 1 < NCHUNK)
        def _():
            pltpu.make_async_copy(chunk_src(g + 1), xb.at[nslot], sem.at[nslot]).start()

        for r in range(CH):
            off = r * C
            acc = xb[slot, pl.ds(off, 16)] * wv[pl.ds(0, 16)]
            for k in range(1, K):
                acc = acc + xb[slot, pl.ds(off + k * 16, 16)] * wv[pl.ds(k * 16, 16)]
            ob[pl.ds((g * CH + r) * 16, 16)] = acc
        return carry

    lax.fori_loop(0, NCHUNK, chunk_body, 0)
    pltpu.sync_copy(ob, p_hbm.at[pl.ds(wid * ROWS_PW * 16, ROWS_PW * 16)])


_sc_cam = functools.partial(
    pl.kernel,
    out_type=jax.ShapeDtypeStruct((R_SC * 16,), jnp.float32),
    mesh=plsc.VectorSubcoreMesh(core_axis_name="c", subcore_axis_name="s"),
    scratch_types=[
        pltpu.VMEM((C,), jnp.float32),
        pltpu.VMEM((2, CH * C), jnp.float32),
        pltpu.VMEM((ROWS_PW * 16,), jnp.float32),
        pltpu.SemaphoreType.DMA((2,)),
    ],
)(_sc_cam_body)


def _reduce_body(p_ref, o_ref):
    r = jnp.sum(p_ref[...], axis=1)
    o_ref[...] = r.reshape(r.shape[0] // 128, 128)


def kernel(conv_input, output, weight):
    x = conv_input.reshape(N * C)
    parts = _sc_cam(x, weight)
    RED_ROWS = 16384
    out = pl.pallas_call(
        _reduce_body,
        grid=(R_SC // RED_ROWS,),
        in_specs=[pl.BlockSpec((RED_ROWS, 16), lambda i: (i, 0))],
        out_specs=pl.BlockSpec((RED_ROWS // 128, 128), lambda i: (i, 0)),
        out_shape=jax.ShapeDtypeStruct((R_SC // 128, 128), jnp.float32),
    )(parts.reshape(R_SC, 16))
    return (out.reshape(B, H, W), output)
